# baseline (speedup 1.0000x reference)
import functools

import jax
import jax.numpy as jnp
from jax import lax
from jax.experimental import pallas as pl
from jax.experimental.pallas import tpu as pltpu
from jax.experimental.pallas import tpu_sc as plsc

NUM_CLASSES = 1000
BATCH = 1024
SEQ = 26
NUM_CORES = 2
NUM_SUBCORES = 16
KCH = 40
NKC = NUM_CLASSES // KCH
GROUPS = BATCH // 16
BUFGROUPS = KCH * BATCH // 16
KDIV_MAGIC = 1639

_mesh = plsc.VectorSubcoreMesh(core_axis_name="c", subcore_axis_name="s")


@functools.partial(
    pl.kernel,
    out_type=jax.ShapeDtypeStruct((SEQ, NUM_CLASSES, BATCH), jnp.float32),
    mesh=_mesh,
    scratch_types=[
        pltpu.VMEM((1, KCH, BATCH), jnp.float32),
        pltpu.VMEM((1, KCH, BATCH), jnp.float32),
        pltpu.VMEM((1, KCH, BATCH), jnp.float32),
        pltpu.VMEM((BATCH,), jnp.int32),
        pltpu.VMEM((BATCH,), jnp.int32),
        pltpu.VMEM((BATCH,), jnp.int32),
        pltpu.SemaphoreType.DMA,
        pltpu.SemaphoreType.DMA,
        pltpu.SemaphoreType.DMA,
    ],
    compiler_params=pltpu.CompilerParams(
        needs_layout_passes=False,
        skip_device_barrier=True,
        disable_bounds_checks=True,
        disable_semaphore_checks=True,
    ),
)
def _onehot_sc(xt_hbm, zeros_hbm, out_hbm,
               buf0, buf1, buf2, col, kcv, relv, sem0, sem1, sem2):
    wid = lax.axis_index("s") * NUM_CORES + lax.axis_index("c")

    @pl.when(wid < SEQ)
    def _():
        s = wid
        z0 = pltpu.async_copy(zeros_hbm, buf0, sem0)
        z2 = pltpu.async_copy(zeros_hbm, buf2, sem2)
        pltpu.sync_copy(xt_hbm.at[pl.ds(s * BATCH, BATCH)], col)

        lane = lax.iota(jnp.int32, 16)
        ones16 = jnp.full((16,), 1.0, jnp.float32)
        zeros16 = jnp.zeros((16,), jnp.float32)
        zeroidx16 = jnp.zeros((16,), jnp.int32)

        def zero_fill(g, _):
            buf1[0, g >> 6, pl.ds((g & 63) * 16, 16)] = zeros16
            return 0

        lax.fori_loop(0, BUFGROUPS, zero_fill, 0, unroll=8)

        def bucket(g, _):
            kv = col[pl.ds(g * 16, 16)]
            kc = (kv * KDIV_MAGIC) >> 16
            kcv[pl.ds(g * 16, 16)] = kc
            relv[pl.ds(g * 16, 16)] = kv - kc * KCH
            return 0

        lax.fori_loop(0, GROUPS, bucket, 0, unroll=4)
        z0.wait()
        z2.wait()

        bufs = (buf0, buf1, buf2)
        sems = (sem0, sem1, sem2)

        def sweep(buf, set_kc, clear_kc):
            def body(g, _):
                kc = kcv[pl.ds(g * 16, 16)]
                rel = relv[pl.ds(g * 16, 16)]
                blane = g * 16 + lane
                if clear_kc is not None:
                    plsc.store_scatter(buf, [zeroidx16, rel, blane],
                                       zeros16, mask=kc == clear_kc)
                plsc.store_scatter(buf, [zeroidx16, rel, blane],
                                   ones16, mask=kc == set_kc)
                return 0

            lax.fori_loop(0, GROUPS, body, 0, unroll=4)

        pending = [None, None, None]
        for kc in range(NKC):
            b = kc % 3
            if pending[b] is not None:
                pending[b].wait()
            sweep(bufs[b], kc, kc - 3 if kc >= 3 else None)
            dst = out_hbm.at[pl.ds(s, 1), pl.ds(kc * KCH, KCH),
                             pl.ds(0, BATCH)]
            pending[b] = pltpu.async_copy(bufs[b], dst, sems[b])
        for b in (0, 1, 2):
            pending[b].wait()


def kernel(x):
    xt = x.astype(jnp.int32).T.reshape(SEQ * BATCH)
    zeros = jnp.zeros((1, KCH, BATCH), jnp.float32)
    y = _onehot_sc(xt, zeros)
    return jnp.transpose(y, (2, 0, 1))


# speedup vs baseline: 1.0828x; 1.0828x over previous
import functools

import jax
import jax.numpy as jnp
from jax import lax
from jax.experimental import pallas as pl
from jax.experimental.pallas import tpu as pltpu
from jax.experimental.pallas import tpu_sc as plsc

NUM_CLASSES = 1000
BATCH = 1024
SEQ = 26
NUM_CORES = 2
NUM_SUBCORES = 16
KCH = 40
NKC = NUM_CLASSES // KCH
GROUPS = BATCH // 16
BUFGROUPS = KCH * BATCH // 16
KDIV_MAGIC = 1639

_mesh = plsc.VectorSubcoreMesh(core_axis_name="c", subcore_axis_name="s")


@functools.partial(
    pl.kernel,
    out_type=jax.ShapeDtypeStruct((SEQ, NUM_CLASSES, BATCH), jnp.float32),
    mesh=_mesh,
    scratch_types=[
        pltpu.VMEM((1, KCH, BATCH), jnp.float32),
        pltpu.VMEM((1, KCH, BATCH), jnp.float32),
        pltpu.VMEM((BATCH,), jnp.int32),
        pltpu.VMEM((BATCH,), jnp.int32),
        pltpu.VMEM((BATCH,), jnp.int32),
        pltpu.SemaphoreType.DMA,
        pltpu.SemaphoreType.DMA,
    ],
    compiler_params=pltpu.CompilerParams(
        needs_layout_passes=False,
        skip_device_barrier=True,
        disable_bounds_checks=True,
        disable_semaphore_checks=True,
    ),
)
def _onehot_sc(xt_hbm, zeros_hbm, out_hbm,
               buf0, buf1, col, kcv, relv, sem0, sem1):
    wid = lax.axis_index("s") * NUM_CORES + lax.axis_index("c")

    @pl.when(wid < SEQ)
    def _():
        s = wid
        z0 = pltpu.async_copy(zeros_hbm, buf0, sem0)
        pltpu.sync_copy(xt_hbm.at[pl.ds(s * BATCH, BATCH)], col)

        lane = lax.iota(jnp.int32, 16)
        ones16 = jnp.full((16,), 1.0, jnp.float32)
        zeros16 = jnp.zeros((16,), jnp.float32)
        zeroidx16 = jnp.zeros((16,), jnp.int32)

        def bucket(g, _):
            kv = col[pl.ds(g * 16, 16)]
            kc = (kv * KDIV_MAGIC) >> 16
            kcv[pl.ds(g * 16, 16)] = kc
            relv[pl.ds(g * 16, 16)] = kv - kc * KCH
            return 0

        lax.fori_loop(0, GROUPS, bucket, 0, unroll=4)
        z0.wait()

        bufs = (buf0, buf1)
        sems = (sem0, sem1)

        def sweep(buf, set_kc, clear_kc):
            def body(g, _):
                kc = kcv[pl.ds(g * 16, 16)]
                rel = relv[pl.ds(g * 16, 16)]
                blane = g * 16 + lane
                if clear_kc is not None:
                    plsc.store_scatter(buf, [zeroidx16, rel, blane],
                                       zeros16, mask=kc == clear_kc)
                plsc.store_scatter(buf, [zeroidx16, rel, blane],
                                   ones16, mask=kc == set_kc)
                return 0

            lax.fori_loop(0, GROUPS, body, 0, unroll=4)

        pending = [None, None]
        sweep(bufs[0], 0, None)
        dst0 = out_hbm.at[pl.ds(s, 1), pl.ds(0, KCH), pl.ds(0, BATCH)]
        pending[0] = pltpu.async_copy(bufs[0], dst0, sems[0])

        def zero_fill(g, _):
            buf1[0, g >> 6, pl.ds((g & 63) * 16, 16)] = zeros16
            return 0

        lax.fori_loop(0, BUFGROUPS, zero_fill, 0, unroll=8)

        for kc in range(1, NKC):
            b = kc % 2
            if pending[b] is not None:
                pending[b].wait()
            sweep(bufs[b], kc, kc - 2 if kc >= 2 else None)
            dst = out_hbm.at[pl.ds(s, 1), pl.ds(kc * KCH, KCH),
                             pl.ds(0, BATCH)]
            pending[b] = pltpu.async_copy(bufs[b], dst, sems[b])
        for b in (0, 1):
            pending[b].wait()


def kernel(x):
    xt = x.astype(jnp.int32).T.reshape(SEQ * BATCH)
    zeros = jnp.zeros((1, KCH, BATCH), jnp.float32)
    y = _onehot_sc(xt, zeros)
    return jnp.transpose(y, (2, 0, 1))


# speedup vs baseline: 1.1130x; 1.0279x over previous
import functools

import jax
import jax.numpy as jnp
from jax import lax
from jax.experimental import pallas as pl
from jax.experimental.pallas import tpu as pltpu
from jax.experimental.pallas import tpu_sc as plsc

NUM_CLASSES = 1000
BATCH = 1024
SEQ = 26
NUM_CORES = 2
NUM_SUBCORES = 16
KCH = 40
NKC = NUM_CLASSES // KCH
GROUPS = BATCH // 16
BUFGROUPS = KCH * BATCH // 16
KDIV_MAGIC = 1639

_mesh = plsc.VectorSubcoreMesh(core_axis_name="c", subcore_axis_name="s")


@functools.partial(
    pl.kernel,
    out_type=jax.ShapeDtypeStruct((SEQ, NUM_CLASSES, BATCH), jnp.float32),
    mesh=_mesh,
    scratch_types=[
        pltpu.VMEM((1, KCH, BATCH), jnp.float32),
        pltpu.VMEM((1, KCH, BATCH), jnp.float32),
        pltpu.VMEM((BATCH,), jnp.int32),
        pltpu.VMEM((BATCH,), jnp.int32),
        pltpu.VMEM((BATCH,), jnp.int32),
        pltpu.SemaphoreType.DMA,
        pltpu.SemaphoreType.DMA,
    ],
    compiler_params=pltpu.CompilerParams(
        needs_layout_passes=False,
        disable_bounds_checks=True,
        disable_semaphore_checks=True,
    ),
)
def _onehot_sc(xt_hbm, zeros_hbm, out_hbm,
               buf0, buf1, col, kcv, relv, sem0, sem1):
    wid = lax.axis_index("s") * NUM_CORES + lax.axis_index("c")

    @pl.when(wid < SEQ)
    def _():
        s = wid
        z0 = pltpu.async_copy(zeros_hbm, buf0, sem0)
        pltpu.sync_copy(xt_hbm.at[pl.ds(s * BATCH, BATCH)], col)

        lane = lax.iota(jnp.int32, 16)
        ones16 = jnp.full((16,), 1.0, jnp.float32)
        zeros16 = jnp.zeros((16,), jnp.float32)
        zeroidx16 = jnp.zeros((16,), jnp.int32)

        def zero_fill(g, _):
            buf1[0, g >> 6, pl.ds((g & 63) * 16, 16)] = zeros16
            return 0

        lax.fori_loop(0, BUFGROUPS, zero_fill, 0, unroll=8)

        def bucket(g, _):
            kv = col[pl.ds(g * 16, 16)]
            kc = (kv * KDIV_MAGIC) >> 16
            kcv[pl.ds(g * 16, 16)] = kc
            relv[pl.ds(g * 16, 16)] = kv - kc * KCH
            return 0

        lax.fori_loop(0, GROUPS, bucket, 0, unroll=4)
        z0.wait()

        bufs = (buf0, buf1)
        sems = (sem0, sem1)

        def sweep(buf, set_kc, clear_kc):
            def body(g, _):
                kc = kcv[pl.ds(g * 16, 16)]
                rel = relv[pl.ds(g * 16, 16)]
                blane = g * 16 + lane
                if clear_kc is not None:
                    plsc.store_scatter(buf, [zeroidx16, rel, blane],
                                       zeros16, mask=kc == clear_kc)
                plsc.store_scatter(buf, [zeroidx16, rel, blane],
                                   ones16, mask=kc == set_kc)
                return 0

            lax.fori_loop(0, GROUPS, body, 0, unroll=4)

        pending = [None, None]
        for kc in range(NKC):
            b = kc % 2
            if pending[b] is not None:
                pending[b].wait()
            sweep(bufs[b], kc, kc - 2 if kc >= 2 else None)
            dst = out_hbm.at[pl.ds(s, 1), pl.ds(kc * KCH, KCH),
                             pl.ds(0, BATCH)]
            pending[b] = pltpu.async_copy(bufs[b], dst, sems[b])
        for b in (0, 1):
            pending[b].wait()


def kernel(x):
    xt = x.astype(jnp.int32).T.reshape(SEQ * BATCH)
    zeros = jnp.zeros((1, KCH, BATCH), jnp.float32)
    y = _onehot_sc(xt, zeros)
    return jnp.transpose(y, (2, 0, 1))
